# initial kernel scaffold (unmeasured)
import jax
import jax.numpy as jnp
from jax import lax
from jax.experimental import pallas as pl
from jax.experimental.pallas import tpu as pltpu


def kernel(x, assign, W1, W2):
    T, D = x.shape
    E, _, F = W1.shape

    assign2 = assign.reshape(1, T).astype(jnp.int32)

    def body(x_ref, a_ref, w1_ref, w2_ref, out_ref,
             xg_ref, ag_ref, send_ref, recv_ref, send_sems, recv_sems):
        my_x = lax.axis_index("x")
        my_y = lax.axis_index("y")
        nbr = (1 - my_x, my_y)

        barrier = pltpu.get_barrier_semaphore()
        pl.semaphore_signal(
            barrier, inc=1, device_id=nbr, device_id_type=pl.DeviceIdType.MESH
        )
        pl.semaphore_wait(barrier, 1)

        xg_ref[0] = x_ref[...]
        ag_ref[0] = a_ref[...]

        rdma_x = pltpu.make_async_remote_copy(
            src_ref=xg_ref.at[0], dst_ref=xg_ref.at[1],
            send_sem=send_sems.at[0], recv_sem=recv_sems.at[0],
            device_id=nbr, device_id_type=pl.DeviceIdType.MESH,
        )
        rdma_a = pltpu.make_async_remote_copy(
            src_ref=ag_ref.at[0], dst_ref=ag_ref.at[1],
            send_sem=send_sems.at[1], recv_sem=recv_sems.at[1],
            device_id=nbr, device_id_type=pl.DeviceIdType.MESH,
        )
        rdma_x.start()
        rdma_a.start()
        rdma_x.wait()
        rdma_a.wait()

        X = jnp.reshape(xg_ref[...], (2 * T, D))
        A = jnp.reshape(ag_ref[...], (2 * T, 1))
        acc = jnp.zeros((2 * T, D), jnp.float32)
        for e in range(E):
            gid = my_x * E + e
            h = jnp.maximum(
                lax.dot_general(
                    X, w1_ref[e],
                    dimension_numbers=(((1,), (0,)), ((), ())),
                    preferred_element_type=jnp.float32,
                ),
                0.0,
            )
            y_e = lax.dot_general(
                h, w2_ref[e],
                dimension_numbers=(((1,), (0,)), ((), ())),
                preferred_element_type=jnp.float32,
            )
            acc = acc + jnp.where(A == gid, y_e, 0.0)

        send_ref[...] = acc[T:, :]
        rdma_o = pltpu.make_async_remote_copy(
            src_ref=send_ref, dst_ref=recv_ref,
            send_sem=send_sems.at[2], recv_sem=recv_sems.at[2],
            device_id=nbr, device_id_type=pl.DeviceIdType.MESH,
        )
        rdma_o.start()
        rdma_o.wait()

        out_ref[...] = acc[:T, :] + recv_ref[...]

    return pl.pallas_call(
        body,
        out_shape=jax.ShapeDtypeStruct((T, D), jnp.float32),
        in_specs=[
            pl.BlockSpec(memory_space=pltpu.VMEM),
            pl.BlockSpec(memory_space=pltpu.VMEM),
            pl.BlockSpec(memory_space=pltpu.VMEM),
            pl.BlockSpec(memory_space=pltpu.VMEM),
        ],
        out_specs=pl.BlockSpec(memory_space=pltpu.VMEM),
        scratch_shapes=[
            pltpu.VMEM((2, T, D), jnp.float32),
            pltpu.VMEM((2, 1, T), jnp.int32),
            pltpu.VMEM((T, D), jnp.float32),
            pltpu.VMEM((T, D), jnp.float32),
            pltpu.SemaphoreType.DMA((3,)),
            pltpu.SemaphoreType.DMA((3,)),
        ],
        compiler_params=pltpu.CompilerParams(collective_id=0),
    )(x, assign2, W1, W2)


# baseline (device time: 43947 ns/iter reference)
import jax
import jax.numpy as jnp
from jax import lax
from jax.experimental import pallas as pl
from jax.experimental.pallas import tpu as pltpu


def kernel(x, assign, W1, W2):
    T, D = x.shape
    E, _, F = W1.shape

    assign2 = assign.reshape(T, 1).astype(jnp.int32)

    def body(x_ref, a_ref, w1_ref, w2_ref, out_ref,
             xg_ref, ag_ref, send_ref, recv_ref, send_sems, recv_sems):
        my_x = lax.axis_index("x")
        my_y = lax.axis_index("y")
        nbr = (1 - my_x, my_y)

        barrier = pltpu.get_barrier_semaphore()
        pl.semaphore_signal(
            barrier, inc=1, device_id=nbr, device_id_type=pl.DeviceIdType.MESH
        )
        pl.semaphore_wait(barrier, 1)

        xg_ref[0] = x_ref[...]
        ag_ref[0] = a_ref[...]

        rdma_x = pltpu.make_async_remote_copy(
            src_ref=xg_ref.at[0], dst_ref=xg_ref.at[1],
            send_sem=send_sems.at[0], recv_sem=recv_sems.at[0],
            device_id=nbr, device_id_type=pl.DeviceIdType.MESH,
        )
        rdma_a = pltpu.make_async_remote_copy(
            src_ref=ag_ref.at[0], dst_ref=ag_ref.at[1],
            send_sem=send_sems.at[1], recv_sem=recv_sems.at[1],
            device_id=nbr, device_id_type=pl.DeviceIdType.MESH,
        )
        rdma_x.start()
        rdma_a.start()
        rdma_x.wait()
        rdma_a.wait()

        X = jnp.reshape(xg_ref[...], (2 * T, D))
        A = jnp.concatenate([ag_ref[0], ag_ref[1]], axis=0)
        acc = jnp.zeros((2 * T, D), jnp.float32)
        for e in range(E):
            gid = my_x * E + e
            h = jnp.maximum(
                lax.dot_general(
                    X, w1_ref[e],
                    dimension_numbers=(((1,), (0,)), ((), ())),
                    preferred_element_type=jnp.float32,
                ),
                0.0,
            )
            y_e = lax.dot_general(
                h, w2_ref[e],
                dimension_numbers=(((1,), (0,)), ((), ())),
                preferred_element_type=jnp.float32,
            )
            acc = acc + jnp.where(A == gid, y_e, 0.0)

        send_ref[...] = acc[T:, :]
        rdma_o = pltpu.make_async_remote_copy(
            src_ref=send_ref, dst_ref=recv_ref,
            send_sem=send_sems.at[2], recv_sem=recv_sems.at[2],
            device_id=nbr, device_id_type=pl.DeviceIdType.MESH,
        )
        rdma_o.start()
        rdma_o.wait()

        out_ref[...] = acc[:T, :] + recv_ref[...]

    return pl.pallas_call(
        body,
        out_shape=jax.ShapeDtypeStruct((T, D), jnp.float32),
        in_specs=[
            pl.BlockSpec(memory_space=pltpu.VMEM),
            pl.BlockSpec(memory_space=pltpu.VMEM),
            pl.BlockSpec(memory_space=pltpu.VMEM),
            pl.BlockSpec(memory_space=pltpu.VMEM),
        ],
        out_specs=pl.BlockSpec(memory_space=pltpu.VMEM),
        scratch_shapes=[
            pltpu.VMEM((2, T, D), jnp.float32),
            pltpu.VMEM((2, T, 1), jnp.int32),
            pltpu.VMEM((T, D), jnp.float32),
            pltpu.VMEM((T, D), jnp.float32),
            pltpu.SemaphoreType.DMA((3,)),
            pltpu.SemaphoreType.DMA((3,)),
        ],
        compiler_params=pltpu.CompilerParams(collective_id=0),
    )(x, assign2, W1, W2)


# device time: 36639 ns/iter; 1.1995x vs baseline; 1.1995x over previous
import jax
import jax.numpy as jnp
from jax import lax
from jax.experimental import pallas as pl
from jax.experimental.pallas import tpu as pltpu

NCHUNK = 2


def kernel(x, assign, W1, W2):
    T, D = x.shape
    E, _, F = W1.shape
    Tc = T // NCHUNK

    assign2 = assign.reshape(T, 1).astype(jnp.int32)

    def moe(X, A, w1_ref, w2_ref, my_x):
        acc = jnp.zeros(X.shape, jnp.float32)
        for e in range(E):
            gid = my_x * E + e
            h = jnp.maximum(
                lax.dot_general(
                    X, w1_ref[e],
                    dimension_numbers=(((1,), (0,)), ((), ())),
                    preferred_element_type=jnp.float32,
                ),
                0.0,
            )
            y_e = lax.dot_general(
                h, w2_ref[e],
                dimension_numbers=(((1,), (0,)), ((), ())),
                preferred_element_type=jnp.float32,
            )
            acc = acc + jnp.where(A == gid, y_e, 0.0)
        return acc

    def body(x_ref, a_ref, w1_ref, w2_ref, out_ref,
             xin_ref, ain_ref, sp_ref, rp_ref, send_sems, recv_sems):
        my_x = lax.axis_index("x")
        my_y = lax.axis_index("y")
        nbr = (1 - my_x, my_y)

        barrier = pltpu.get_barrier_semaphore()
        pl.semaphore_signal(
            barrier, inc=1, device_id=nbr, device_id_type=pl.DeviceIdType.MESH
        )
        pl.semaphore_wait(barrier, 1)

        def remote(src, dst, sem_idx):
            return pltpu.make_async_remote_copy(
                src_ref=src, dst_ref=dst,
                send_sem=send_sems.at[sem_idx], recv_sem=recv_sems.at[sem_idx],
                device_id=nbr, device_id_type=pl.DeviceIdType.MESH,
            )

        rdma_a = remote(a_ref, ain_ref, 0)
        rdma_a.start()
        rdma_x = []
        for c in range(NCHUNK):
            r = remote(x_ref.at[pl.ds(c * Tc, Tc), :],
                       xin_ref.at[pl.ds(c * Tc, Tc), :], 1 + c)
            r.start()
            rdma_x.append(r)

        acc_mine = moe(x_ref[...], a_ref[...], w1_ref, w2_ref, my_x)

        rdma_a.wait_recv()
        rdma_p = []
        for c in range(NCHUNK):
            rdma_x[c].wait_recv()
            pc = moe(xin_ref[pl.ds(c * Tc, Tc), :],
                     ain_ref[pl.ds(c * Tc, Tc), :], w1_ref, w2_ref, my_x)
            sp_ref[pl.ds(c * Tc, Tc), :] = pc
            r = remote(sp_ref.at[pl.ds(c * Tc, Tc), :],
                       rp_ref.at[pl.ds(c * Tc, Tc), :], 1 + NCHUNK + c)
            r.start()
            rdma_p.append(r)

        for r in rdma_p:
            r.wait_recv()
        out_ref[...] = acc_mine + rp_ref[...]

        rdma_a.wait_send()
        for r in rdma_x + rdma_p:
            r.wait_send()

    return pl.pallas_call(
        body,
        out_shape=jax.ShapeDtypeStruct((T, D), jnp.float32),
        in_specs=[
            pl.BlockSpec(memory_space=pltpu.VMEM),
            pl.BlockSpec(memory_space=pltpu.VMEM),
            pl.BlockSpec(memory_space=pltpu.VMEM),
            pl.BlockSpec(memory_space=pltpu.VMEM),
        ],
        out_specs=pl.BlockSpec(memory_space=pltpu.VMEM),
        scratch_shapes=[
            pltpu.VMEM((T, D), jnp.float32),
            pltpu.VMEM((T, 1), jnp.int32),
            pltpu.VMEM((T, D), jnp.float32),
            pltpu.VMEM((T, D), jnp.float32),
            pltpu.SemaphoreType.DMA((1 + 2 * NCHUNK,)),
            pltpu.SemaphoreType.DMA((1 + 2 * NCHUNK,)),
        ],
        compiler_params=pltpu.CompilerParams(collective_id=0),
    )(x, assign2, W1, W2)


# device time: 29486 ns/iter; 1.4904x vs baseline; 1.2426x over previous
import jax
import jax.numpy as jnp
from jax import lax
from jax.experimental import pallas as pl
from jax.experimental.pallas import tpu as pltpu

NCH = 2


def kernel(x, assign, W1, W2):
    T, D = x.shape
    E, _, F = W1.shape
    Th = T // 2
    Tc = Th // NCH

    assign2 = assign.reshape(T, 1).astype(jnp.int32)

    def moe(X, A, w1_ref, w2_ref, my_x):
        acc = jnp.zeros(X.shape, jnp.float32)
        for e in range(E):
            gid = my_x * E + e
            h = jnp.maximum(
                lax.dot_general(
                    X, w1_ref[e],
                    dimension_numbers=(((1,), (0,)), ((), ())),
                    preferred_element_type=jnp.float32,
                ),
                0.0,
            )
            y_e = lax.dot_general(
                h, w2_ref[e],
                dimension_numbers=(((1,), (0,)), ((), ())),
                preferred_element_type=jnp.float32,
            )
            acc = acc + jnp.where(A == gid, y_e, 0.0)
        return acc

    def body(x_ref, a_ref, w1_ref, w2_ref, out_ref,
             xin_ref, ain_ref, sp_ref, cp_ref, send_sems, recv_sems):
        my_x = lax.axis_index("x")
        my_y = lax.axis_index("y")
        xnbr = (1 - my_x, my_y)
        ynbr = (my_x, 1 - my_y)
        h0 = my_y * Th

        barrier = pltpu.get_barrier_semaphore()
        for nbr in (xnbr, ynbr):
            pl.semaphore_signal(
                barrier, inc=1, device_id=nbr,
                device_id_type=pl.DeviceIdType.MESH,
            )
        pl.semaphore_wait(barrier, 2)

        def remote(src, dst, sem_idx, dev):
            return pltpu.make_async_remote_copy(
                src_ref=src, dst_ref=dst,
                send_sem=send_sems.at[sem_idx], recv_sem=recv_sems.at[sem_idx],
                device_id=dev, device_id_type=pl.DeviceIdType.MESH,
            )

        rdma_a = remote(a_ref.at[pl.ds(h0, Th), :], ain_ref, 0, xnbr)
        rdma_a.start()
        rdma_x = []
        for c in range(NCH):
            r = remote(x_ref.at[pl.ds(h0 + c * Tc, Tc), :],
                       xin_ref.at[pl.ds(c * Tc, Tc), :], 1 + c, xnbr)
            r.start()
            rdma_x.append(r)

        acc_mine = moe(x_ref[...], a_ref[...], w1_ref, w2_ref, my_x)

        rdma_a.wait_recv()
        rdma_c = []
        for c in range(NCH):
            rdma_x[c].wait_recv()
            pc = moe(xin_ref[pl.ds(c * Tc, Tc), :],
                     ain_ref[pl.ds(c * Tc, Tc), :], w1_ref, w2_ref, my_x)
            sp_ref[pl.ds(c * Tc, Tc), :] = pc
            r = remote(sp_ref.at[pl.ds(c * Tc, Tc), :],
                       cp_ref.at[pl.ds(h0 + c * Tc, Tc), :],
                       1 + NCH + c, xnbr)
            r.start()
            rdma_c.append(r)

        rdma_d = []
        for c in range(NCH):
            rdma_c[c].wait_recv()
            r = remote(cp_ref.at[pl.ds(h0 + c * Tc, Tc), :],
                       cp_ref.at[pl.ds(h0 + c * Tc, Tc), :],
                       1 + 2 * NCH + c, ynbr)
            r.start()
            rdma_d.append(r)

        for r in rdma_d:
            r.wait_recv()
        out_ref[...] = acc_mine + cp_ref[...]

        rdma_a.wait_send()
        for r in rdma_x + rdma_c + rdma_d:
            r.wait_send()

    return pl.pallas_call(
        body,
        out_shape=jax.ShapeDtypeStruct((T, D), jnp.float32),
        in_specs=[
            pl.BlockSpec(memory_space=pltpu.VMEM),
            pl.BlockSpec(memory_space=pltpu.VMEM),
            pl.BlockSpec(memory_space=pltpu.VMEM),
            pl.BlockSpec(memory_space=pltpu.VMEM),
        ],
        out_specs=pl.BlockSpec(memory_space=pltpu.VMEM),
        scratch_shapes=[
            pltpu.VMEM((Th, D), jnp.float32),
            pltpu.VMEM((Th, 1), jnp.int32),
            pltpu.VMEM((Th, D), jnp.float32),
            pltpu.VMEM((T, D), jnp.float32),
            pltpu.SemaphoreType.DMA((1 + 3 * NCH,)),
            pltpu.SemaphoreType.DMA((1 + 3 * NCH,)),
        ],
        compiler_params=pltpu.CompilerParams(collective_id=0),
    )(x, assign2, W1, W2)
